# baseline (speedup 1.0000x reference)
import functools
import math

import jax
import jax.numpy as jnp
from jax.experimental import pallas as pl
from jax.experimental.pallas import tpu as pltpu

D_MODEL_ = 1024
N_HEADS_ = 16
N_SEL_ = 64
IDX_DIM_ = 32
D_HEAD_ = D_MODEL_ // N_HEADS_

_INT_MIN = -2147483648
_INT_MAX = 2147483647


def _sortable_key(x):
    bits = jax.lax.bitcast_convert_type(x, jnp.int32)
    return jnp.where(bits < 0, bits ^ jnp.int32(0x7FFFFFFF), bits)


def _floor_avg(a, b):
    return (a & b) + ((a ^ b) >> 1)


def _attn_body(x_ref, wq_ref, wk_ref, wv_ref, wo_ref, wqi_ref, wki_ref,
               out_ref, u_scr, st_scr):
    h = pl.program_id(0)
    f32 = jnp.float32
    i32 = jnp.int32

    x = x_ref[...]
    q = jnp.dot(x, wq_ref[0], preferred_element_type=f32)
    k = jnp.dot(x, wk_ref[0], preferred_element_type=f32)
    v = jnp.dot(x, wv_ref[0], preferred_element_type=f32)

    qi = jnp.dot(q, wqi_ref[...], preferred_element_type=f32)
    ki = jnp.dot(k, wki_ref[...], preferred_element_type=f32)
    sit = jnp.dot(ki, qi.T, preferred_element_type=f32)
    sit = sit * f32(1.0 / math.sqrt(IDX_DIM_))

    u_scr[...] = _sortable_key(sit)

    t = sit.shape[0]
    lo0 = jnp.full((1, t), _INT_MIN, dtype=i32)
    hi0 = jnp.full((1, t), _INT_MAX, dtype=i32)

    ones8 = jnp.ones((8, t), dtype=f32)
    qs = q * f32(1.0 / math.sqrt(D_HEAD_))

    n_chunk = t // 128

    lo, hi = lo0, hi0
    for i in range(32):
        mid = _floor_avg(lo, hi)
        ind = (u_scr[...] >= mid).astype(f32)
        cnt = jnp.dot(ones8, ind, preferred_element_type=f32)[0:1]
        ge = cnt >= f32(N_SEL_)
        if i < n_chunk:
            qc = qs[i * 128:(i + 1) * 128, :]
            st_scr[:, i * 128:(i + 1) * 128] = jnp.dot(
                k, qc.T, preferred_element_type=f32)
        lo = jnp.where(ge, mid, lo)
        hi = jnp.where(ge, hi, mid)
    tau = lo

    m = jnp.max(jnp.where(u_scr[...] >= tau, st_scr[...], f32(-jnp.inf)),
                axis=0, keepdims=True)
    st_scr[...] = jnp.where(u_scr[...] >= tau,
                            jnp.exp(st_scr[...] - m), f32(0.0))
    den = jnp.dot(ones8, st_scr[...], preferred_element_type=f32)[0:1]

    o = jax.lax.dot_general(st_scr[...], v, (((0,), (0,)), ((), ())),
                            preferred_element_type=f32)
    o = o * (f32(1.0) / den.T)
    contrib = jnp.dot(o, wo_ref[0], preferred_element_type=f32)

    @pl.when(h == 0)
    def _():
        out_ref[...] = contrib

    @pl.when(h != 0)
    def _():
        out_ref[...] += contrib


@jax.jit
def kernel(hidden_states, Wq, Wk, Wv, Wo, Wq_idx, Wk_idx):
    b, t, d = hidden_states.shape
    x = hidden_states.reshape(t, d)
    dh = D_HEAD_

    wq_h = Wq.reshape(d, N_HEADS_, dh).transpose(1, 0, 2)
    wk_h = Wk.reshape(d, N_HEADS_, dh).transpose(1, 0, 2)
    wv_h = Wv.reshape(d, N_HEADS_, dh).transpose(1, 0, 2)
    wo_h = Wo.reshape(N_HEADS_, dh, d)

    out = pl.pallas_call(
        _attn_body,
        grid=(N_HEADS_,),
        in_specs=[
            pl.BlockSpec((t, d), lambda h: (0, 0)),
            pl.BlockSpec((1, d, dh), lambda h: (h, 0, 0)),
            pl.BlockSpec((1, d, dh), lambda h: (h, 0, 0)),
            pl.BlockSpec((1, d, dh), lambda h: (h, 0, 0)),
            pl.BlockSpec((1, dh, d), lambda h: (h, 0, 0)),
            pl.BlockSpec((dh, IDX_DIM_), lambda h: (0, 0)),
            pl.BlockSpec((dh, IDX_DIM_), lambda h: (0, 0)),
        ],
        out_specs=pl.BlockSpec((t, d), lambda h: (0, 0)),
        out_shape=jax.ShapeDtypeStruct((t, d), jnp.float32),
        scratch_shapes=[pltpu.VMEM((t, t), jnp.int32),
                        pltpu.VMEM((t, t), jnp.float32)],
        compiler_params=pltpu.CompilerParams(
            dimension_semantics=("arbitrary",),
            vmem_limit_bytes=100 * 1024 * 1024),
    )(x, wq_h, wk_h, wv_h, wo_h, Wq_idx, Wk_idx)
    return out.reshape(b, t, d)


# speedup vs baseline: 1.2215x; 1.2215x over previous
import math

import jax
import jax.numpy as jnp
from jax.experimental import pallas as pl
from jax.experimental.pallas import tpu as pltpu

D_MODEL_ = 1024
N_HEADS_ = 16
N_SEL_ = 64
IDX_DIM_ = 32
D_HEAD_ = D_MODEL_ // N_HEADS_

_INT_MIN = -2147483648
_INT_MAX = 2147483647


def _sortable_key(x):
    bits = jax.lax.bitcast_convert_type(x, jnp.int32)
    return jnp.where(bits < 0, bits ^ jnp.int32(0x7FFFFFFF), bits)


def _floor_avg(a, b):
    return (a & b) + ((a ^ b) >> 1)


def _attn_body(x_ref, wq_ref, wk_ref, wv_ref, wo_ref, wqi_ref, wki_ref,
               out_ref, u_scr):
    h = pl.program_id(0)
    f32 = jnp.float32
    i32 = jnp.int32

    x = x_ref[...]
    q = jnp.dot(x, wq_ref[0], preferred_element_type=f32)
    k = jnp.dot(x, wk_ref[0], preferred_element_type=f32)
    v = jnp.dot(x, wv_ref[0], preferred_element_type=f32)

    qi = jnp.dot(q, wqi_ref[...], preferred_element_type=f32)
    ki = jnp.dot(k, wki_ref[...], preferred_element_type=f32)
    sit = jnp.dot(ki, qi.T, preferred_element_type=f32)
    sit = sit * f32(1.0 / math.sqrt(IDX_DIM_))

    u_scr[...] = _sortable_key(sit)

    t = sit.shape[0]
    lo0 = jnp.full((1, t), _INT_MIN, dtype=i32)
    hi0 = jnp.full((1, t), _INT_MAX, dtype=i32)

    ones8 = jnp.ones((8, t), dtype=f32)

    def cond(carry):
        _, _, _, i, conv = carry
        return jnp.logical_and(i < 32, jnp.logical_not(conv))

    def bisect(carry):
        lo, hi, cnt_lo, i, _ = carry
        mid = _floor_avg(lo, hi)
        ind = (u_scr[...] >= mid).astype(f32)
        cnt = jnp.dot(ones8, ind, preferred_element_type=f32)[0:1]
        ge = cnt >= f32(N_SEL_)
        lo = jnp.where(ge, mid, lo)
        hi = jnp.where(ge, hi, mid)
        cnt_lo = jnp.where(ge, cnt, cnt_lo)
        conv = jnp.all(cnt_lo == f32(N_SEL_))
        return lo, hi, cnt_lo, i + 1, conv

    tau, _, _, _, _ = jax.lax.while_loop(
        cond, bisect,
        (lo0, hi0, jnp.full((1, t), t, dtype=f32), jnp.int32(0),
         jnp.bool_(False)))
    maskt = u_scr[...] >= tau

    st = jnp.dot(k, q.T, preferred_element_type=f32) * f32(1.0 / math.sqrt(D_HEAD_))
    st = jnp.where(maskt, st, f32(-jnp.inf))
    m = jnp.max(st, axis=0, keepdims=True)
    p = jnp.exp(st - m)
    den = jnp.sum(p, axis=0, keepdims=True)

    o = jax.lax.dot_general(p, v, (((0,), (0,)), ((), ())),
                            preferred_element_type=f32)
    o = o * (f32(1.0) / den.T)
    contrib = jnp.dot(o, wo_ref[0], preferred_element_type=f32)

    @pl.when(h == 0)
    def _():
        out_ref[...] = contrib

    @pl.when(h != 0)
    def _():
        out_ref[...] += contrib


@jax.jit
def kernel(hidden_states, Wq, Wk, Wv, Wo, Wq_idx, Wk_idx):
    b, t, d = hidden_states.shape
    x = hidden_states.reshape(t, d)
    dh = D_HEAD_

    wq_h = Wq.reshape(d, N_HEADS_, dh).transpose(1, 0, 2)
    wk_h = Wk.reshape(d, N_HEADS_, dh).transpose(1, 0, 2)
    wv_h = Wv.reshape(d, N_HEADS_, dh).transpose(1, 0, 2)
    wo_h = Wo.reshape(N_HEADS_, dh, d)

    out = pl.pallas_call(
        _attn_body,
        grid=(N_HEADS_,),
        in_specs=[
            pl.BlockSpec((t, d), lambda h: (0, 0)),
            pl.BlockSpec((1, d, dh), lambda h: (h, 0, 0)),
            pl.BlockSpec((1, d, dh), lambda h: (h, 0, 0)),
            pl.BlockSpec((1, d, dh), lambda h: (h, 0, 0)),
            pl.BlockSpec((1, dh, d), lambda h: (h, 0, 0)),
            pl.BlockSpec((dh, IDX_DIM_), lambda h: (0, 0)),
            pl.BlockSpec((dh, IDX_DIM_), lambda h: (0, 0)),
        ],
        out_specs=pl.BlockSpec((t, d), lambda h: (0, 0)),
        out_shape=jax.ShapeDtypeStruct((t, d), jnp.float32),
        scratch_shapes=[pltpu.VMEM((t, t), jnp.int32)],
        compiler_params=pltpu.CompilerParams(
            dimension_semantics=("arbitrary",)),
    )(x, wq_h, wk_h, wv_h, wo_h, Wq_idx, Wk_idx)
    return out.reshape(b, t, d)
